# baseline (speedup 1.0000x reference)
import jax
import jax.numpy as jnp
from jax.experimental import pallas as pl
from jax.experimental.pallas import tpu as pltpu

N, D, H, O, E = 4096, 1024, 2048, 1024, 3
BN = 512
NB = N // BN
LW = 8


def _moe_kernel(x_ref, gw1_ref, gb1_ref, gw2_ref, gb2_ref, gw3_ref, gb3_ref,
                ew1_ref, eb1_ref, ew2_ref, eb2_ref, out_ref, gates_ref,
                acc_scr, wg_scr):
    e = pl.program_id(0)
    b = pl.program_id(1)
    xb = x_ref[...]
    lane = jax.lax.broadcasted_iota(jnp.int32, (BN, 128), 1)

    @pl.when(e == 0)
    def _gating():
        h1 = jax.nn.relu(
            jnp.dot(xb, gw1_ref[...], preferred_element_type=jnp.float32)
            + gb1_ref[...])
        h2 = jax.nn.relu(
            jnp.dot(h1, gw2_ref[...], preferred_element_type=jnp.float32)
            + gb2_ref[...])
        logits = jnp.dot(h2, gw3_ref[...],
                         preferred_element_type=jnp.float32) + gb3_ref[...]
        gates = jax.nn.softmax(logits, axis=-1)
        g0, g1, g2 = gates[:, 0], gates[:, 1], gates[:, 2]
        drop2 = (g2 <= g0) & (g2 <= g1)
        drop1 = (~drop2) & (g1 <= g0) & (g1 <= g2)
        drop0 = (~drop2) & (~drop1)
        gmin = jnp.where(drop2, g2, jnp.where(drop1, g1, g0))
        denom = (g0 + g1 + g2) - gmin
        w0 = jnp.where(drop0, 0.0, g0) / denom
        w1 = jnp.where(drop1, 0.0, g1) / denom
        w2 = jnp.where(drop2, 0.0, g2) / denom
        buf = (g0[:, None] * (lane == 0) + g1[:, None] * (lane == 1)
               + g2[:, None] * (lane == 2)
               + w0[:, None] * (lane == LW) + w1[:, None] * (lane == LW + 1)
               + w2[:, None] * (lane == LW + 2))
        wg_scr[b] = buf

    blk = wg_scr[b]
    gates_ref[...] = blk[:, :E]
    we = jnp.sum(jnp.where(lane == LW + e, blk, 0.0), axis=1, keepdims=True)

    h = jax.nn.relu(
        jnp.dot(xb.astype(jnp.bfloat16), ew1_ref[0],
                preferred_element_type=jnp.float32) + eb1_ref[0])
    o = jnp.dot(h.astype(jnp.bfloat16), ew2_ref[0],
                preferred_element_type=jnp.float32) + eb2_ref[0]
    contrib = we * o
    base = b * BN

    @pl.when(e == 0)
    def _init():
        acc_scr[pl.ds(base, BN), :] = contrib

    @pl.when((e > 0) & (e < E - 1))
    def _accum():
        acc_scr[pl.ds(base, BN), :] = acc_scr[pl.ds(base, BN), :] + contrib

    @pl.when(e == E - 1)
    def _final():
        out_ref[...] = acc_scr[pl.ds(base, BN), :] + contrib


def kernel(x, gw1, gb1, gw2, gb2, gw3, gb3, ew1, eb1, ew2, eb2):
    ew1 = ew1.astype(jnp.bfloat16)
    ew2 = ew2.astype(jnp.bfloat16)
    eb1 = eb1.reshape(E, 1, H)
    eb2 = eb2.reshape(E, 1, O)
    gb1 = gb1.reshape(1, -1)
    gb2 = gb2.reshape(1, -1)
    gb3 = gb3.reshape(1, -1)

    grid = (E, NB)
    full = lambda e, b: (0, 0)
    out, gates = pl.pallas_call(
        _moe_kernel,
        grid=grid,
        in_specs=[
            pl.BlockSpec((BN, D), lambda e, b: (b, 0)),
            pl.BlockSpec((D, 256), full),
            pl.BlockSpec((1, 256), full),
            pl.BlockSpec((256, 128), full),
            pl.BlockSpec((1, 128), full),
            pl.BlockSpec((128, E), full),
            pl.BlockSpec((1, E), full),
            pl.BlockSpec((1, D, H), lambda e, b: (e, 0, 0)),
            pl.BlockSpec((1, 1, H), lambda e, b: (e, 0, 0)),
            pl.BlockSpec((1, H, O), lambda e, b: (e, 0, 0)),
            pl.BlockSpec((1, 1, O), lambda e, b: (e, 0, 0)),
        ],
        out_specs=[
            pl.BlockSpec((BN, O),
                         lambda e, b: (jnp.where(e == E - 1, b, 0), 0)),
            pl.BlockSpec((BN, E), lambda e, b: (b, 0)),
        ],
        out_shape=[
            jax.ShapeDtypeStruct((N, O), jnp.float32),
            jax.ShapeDtypeStruct((N, E), jnp.float32),
        ],
        scratch_shapes=[
            pltpu.VMEM((N, O), jnp.float32),
            pltpu.VMEM((NB, BN, 128), jnp.float32),
        ],
    )(x, gw1, gb1, gw2, gb2, gw3, gb3, ew1, eb1, ew2, eb2)
    return out, gates


# speedup vs baseline: 1.1439x; 1.1439x over previous
import jax
import jax.numpy as jnp
from jax.experimental import pallas as pl
from jax.experimental.pallas import tpu as pltpu

N, D, H, O, E = 4096, 1024, 2048, 1024, 3
BN = 256


def _moe_kernel(x_ref, gw1_ref, gb1_ref, gw2_ref, gb2_ref, gw3_ref, gb3_ref,
                ew1_ref, eb1_ref, ew2_ref, eb2_ref, out_ref, gates_ref,
                x16_scr, h16_scr):
    xb = x_ref[...]

    h1 = jax.nn.relu(
        jnp.dot(xb, gw1_ref[...], preferred_element_type=jnp.float32)
        + gb1_ref[...])
    h2 = jax.nn.relu(
        jnp.dot(h1, gw2_ref[...], preferred_element_type=jnp.float32)
        + gb2_ref[...])
    logits = jnp.dot(h2, gw3_ref[...], preferred_element_type=jnp.float32) \
        + gb3_ref[...]
    gates = jax.nn.softmax(logits, axis=-1)
    gates_ref[...] = gates

    g0, g1, g2 = gates[:, 0], gates[:, 1], gates[:, 2]
    drop2 = (g2 <= g0) & (g2 <= g1)
    drop1 = (~drop2) & (g1 <= g0) & (g1 <= g2)
    drop0 = (~drop2) & (~drop1)
    gmin = jnp.where(drop2, g2, jnp.where(drop1, g1, g0))
    denom = (g0 + g1 + g2) - gmin
    w0 = jnp.where(drop0, 0.0, g0) / denom
    w1 = jnp.where(drop1, 0.0, g1) / denom
    w2 = jnp.where(drop2, 0.0, g2) / denom

    x16_scr[...] = xb.astype(jnp.bfloat16)
    acc = jnp.zeros((xb.shape[0], O), jnp.float32)
    for e, we in ((0, w0), (1, w1), (2, w2)):
        h = jax.nn.relu(
            jnp.dot(x16_scr[...], ew1_ref[e],
                    preferred_element_type=jnp.float32)
            + eb1_ref[e])
        h16_scr[...] = h.astype(jnp.bfloat16)
        o = jnp.dot(h16_scr[...], ew2_ref[e],
                    preferred_element_type=jnp.float32) + eb2_ref[e]
        acc = acc + we[:, None] * o
    out_ref[...] = acc


def kernel(x, gw1, gb1, gw2, gb2, gw3, gb3, ew1, eb1, ew2, eb2):
    ew1 = ew1.astype(jnp.bfloat16)
    ew2 = ew2.astype(jnp.bfloat16)
    eb1 = eb1.astype(jnp.bfloat16)
    gb1 = gb1.reshape(1, -1)
    gb2 = gb2.reshape(1, -1)
    gb3 = gb3.reshape(1, -1)

    grid = (N // BN,)
    full = lambda i: (0, 0)
    full3 = lambda i: (0, 0, 0)
    out, gates = pl.pallas_call(
        _moe_kernel,
        grid=grid,
        in_specs=[
            pl.BlockSpec((BN, D), lambda i: (i, 0)),
            pl.BlockSpec((D, 256), full),
            pl.BlockSpec((1, 256), full),
            pl.BlockSpec((256, 128), full),
            pl.BlockSpec((1, 128), full),
            pl.BlockSpec((128, E), full),
            pl.BlockSpec((1, E), full),
            pl.BlockSpec((E, D, H), full3),
            pl.BlockSpec((E, H), full),
            pl.BlockSpec((E, H, O), full3),
            pl.BlockSpec((E, O), full),
        ],
        out_specs=[
            pl.BlockSpec((BN, O), lambda i: (i, 0)),
            pl.BlockSpec((BN, E), lambda i: (i, 0)),
        ],
        out_shape=[
            jax.ShapeDtypeStruct((N, O), jnp.float32),
            jax.ShapeDtypeStruct((N, E), jnp.float32),
        ],
        scratch_shapes=[
            pltpu.VMEM((BN, D), jnp.bfloat16),
            pltpu.VMEM((BN, H), jnp.bfloat16),
        ],
    )(x, gw1, gb1, gw2, gb2, gw3, gb3, ew1, eb1, ew2, eb2)
    return out, gates


# speedup vs baseline: 1.1543x; 1.0091x over previous
import jax
import jax.numpy as jnp
from jax.experimental import pallas as pl
from jax.experimental.pallas import tpu as pltpu

N, D, H, O, E = 4096, 1024, 2048, 1024, 3
BN = 256


def _moe_kernel(x_ref, gw1_ref, gb1_ref, gw2_ref, gb2_ref, gw3_ref, gb3_ref,
                ew1_ref, eb1_ref, ew2_ref, eb2_ref, out_ref, gates_ref):
    xb = x_ref[...]

    h1 = jax.nn.relu(
        jnp.dot(xb, gw1_ref[...], preferred_element_type=jnp.float32)
        + gb1_ref[...])
    h2 = jax.nn.relu(
        jnp.dot(h1, gw2_ref[...], preferred_element_type=jnp.float32)
        + gb2_ref[...])
    logits = jnp.dot(h2, gw3_ref[...], preferred_element_type=jnp.float32) \
        + gb3_ref[...]
    gates = jax.nn.softmax(logits, axis=-1)
    gates_ref[...] = gates

    g0, g1, g2 = gates[:, 0], gates[:, 1], gates[:, 2]
    drop2 = (g2 <= g0) & (g2 <= g1)
    drop1 = (~drop2) & (g1 <= g0) & (g1 <= g2)
    drop0 = (~drop2) & (~drop1)
    gmin = jnp.where(drop2, g2, jnp.where(drop1, g1, g0))
    denom = (g0 + g1 + g2) - gmin
    w0 = jnp.where(drop0, 0.0, g0) / denom
    w1 = jnp.where(drop1, 0.0, g1) / denom
    w2 = jnp.where(drop2, 0.0, g2) / denom

    xb16 = xb.astype(jnp.bfloat16)
    acc = jnp.zeros((xb.shape[0], O), jnp.float32)
    for e, we in ((0, w0), (1, w1), (2, w2)):
        h = jax.nn.relu(
            jnp.dot(xb16, ew1_ref[e], preferred_element_type=jnp.float32)
            + eb1_ref[e])
        o = jnp.dot(h.astype(jnp.bfloat16), ew2_ref[e],
                    preferred_element_type=jnp.float32) + eb2_ref[e]
        acc = acc + we[:, None] * o
    out_ref[...] = acc


def kernel(x, gw1, gb1, gw2, gb2, gw3, gb3, ew1, eb1, ew2, eb2):
    ew1 = ew1.astype(jnp.bfloat16)
    ew2 = ew2.astype(jnp.bfloat16)
    eb1 = eb1.astype(jnp.bfloat16)
    gb1 = gb1.reshape(1, -1)
    gb2 = gb2.reshape(1, -1)
    gb3 = gb3.reshape(1, -1)

    grid = (N // BN,)
    full = lambda i: (0, 0)
    full3 = lambda i: (0, 0, 0)
    out, gates = pl.pallas_call(
        _moe_kernel,
        grid=grid,
        in_specs=[
            pl.BlockSpec((BN, D), lambda i: (i, 0)),
            pl.BlockSpec((D, 256), full),
            pl.BlockSpec((1, 256), full),
            pl.BlockSpec((256, 128), full),
            pl.BlockSpec((1, 128), full),
            pl.BlockSpec((128, E), full),
            pl.BlockSpec((1, E), full),
            pl.BlockSpec((E, D, H), full3),
            pl.BlockSpec((E, H), full),
            pl.BlockSpec((E, H, O), full3),
            pl.BlockSpec((E, O), full),
        ],
        out_specs=[
            pl.BlockSpec((BN, O), lambda i: (i, 0)),
            pl.BlockSpec((BN, E), lambda i: (i, 0)),
        ],
        out_shape=[
            jax.ShapeDtypeStruct((N, O), jnp.float32),
            jax.ShapeDtypeStruct((N, E), jnp.float32),
        ],
        compiler_params=pltpu.CompilerParams(
            vmem_limit_bytes=128 * 1024 * 1024),
    )(x, gw1, gb1, gw2, gb2, gw3, gb3, ew1, eb1, ew2, eb2)
    return out, gates
